# baseline (speedup 1.0000x reference)
import functools

import jax
import jax.numpy as jnp
from jax import lax
from jax.experimental import pallas as pl
from jax.experimental.pallas import tpu as pltpu
from jax.experimental.pallas import tpu_sc as plsc

N = 10000
E = 320000
NTILES = 32
EP = 327680
ET = EP // NTILES
NACC = 10112
R = 2000
GRID = N // R


def _chunkw(F):
    return [32] if F == 32 else [64] * (F // 64)


@functools.cache
def _segsum_fn(F):
    NDMA = 4
    NPAIR = ET // 1024

    mesh = plsc.VectorSubcoreMesh(
        core_axis_name="c", subcore_axis_name="s", num_cores=2)

    @functools.partial(
        pl.kernel,
        out_type=jax.ShapeDtypeStruct((2, N, F), jnp.float32),
        mesh=mesh,
        compiler_params=pltpu.CompilerParams(use_tc_tiling_on_sc=False),
        scratch_types=[
            pltpu.VMEM((4, 128), jnp.int32),
            pltpu.VMEM((4, 128), jnp.int32),
            pltpu.VMEM((4, 128), jnp.int32),
            pltpu.VMEM((4, 128), jnp.int32),
            pltpu.VMEM((512, F), jnp.float32),
            pltpu.VMEM((512, F), jnp.float32),
            pltpu.VMEM_SHARED((NACC, F), jnp.float32),
            pltpu.SemaphoreType.DMA,
            pltpu.SemaphoreType.DMA,
        ],
    )
    def seg(z_hbm, src_hbm, dst_hbm, zeros_hbm, u_hbm,
            srcA, dstA, srcB, dstB, rowsA, rowsB, acc, gsem, ssem):
        c = lax.axis_index("c")
        s = lax.axis_index("s")
        w = c * 16 + s
        pltpu.sync_copy(zeros_hbm.at[pl.ds(s * 632, 632)],
                        acc.at[pl.ds(s * 632, 632)])
        plsc.subcore_barrier()

        def gather_chunk(ci, sv, dv, rows):
            pltpu.sync_copy(src_hbm.at[w, ci], sv)
            pltpu.sync_copy(dst_hbm.at[w, ci], dv)
            return [pltpu.async_copy(z_hbm.at[sv.at[j]],
                                     rows.at[pl.ds(j * 128, 128)], gsem)
                    for j in range(NDMA)]

        def scatter_chunk(dv, rows):
            for j in range(NDMA):
                pltpu.make_async_copy(rows.at[pl.ds(j * 128, 128)],
                                      acc.at[dv.at[j]], ssem).start(add=True)

        def drain_scatters(dv, rows):
            for j in range(NDMA):
                pltpu.make_async_copy(rows.at[pl.ds(j * 128, 128)],
                                      acc.at[dv.at[j]], ssem).wait()

        def body(i, carry):
            ga = gather_chunk(2 * i, srcA, dstA, rowsA)

            @pl.when(i > 0)
            def _():
                drain_scatters(dstB, rowsB)

            for cp in ga:
                cp.wait()
            scatter_chunk(dstA, rowsA)
            gb = gather_chunk(2 * i + 1, srcB, dstB, rowsB)
            for cp in gb:
                cp.wait()
            drain_scatters(dstA, rowsA)
            scatter_chunk(dstB, rowsB)
            return carry

        lax.fori_loop(0, NPAIR, body, 0)
        drain_scatters(dstB, rowsB)
        plsc.subcore_barrier()
        pltpu.sync_copy(acc.at[pl.ds(s * 624, 624)],
                        u_hbm.at[c, pl.ds(s * 624, 624)])

        @pl.when(s == 15)
        def _():
            pltpu.sync_copy(acc.at[pl.ds(9984, 16)],
                            u_hbm.at[c, pl.ds(9984, 16)])

    return seg


def _segsum(zs, edges):
    src3, dst3, zeros = edges
    return [_segsum_fn(z.shape[1])(z, src3, dst3, zeros[:, :z.shape[1]])
            for z in zs]


def _rows(F):
    return pl.BlockSpec((R, F), lambda i: (i, 0))


def _rows3(F):
    return pl.BlockSpec((2, R, F), lambda i: (0, i, 0))


def _whole(a, b):
    return pl.BlockSpec((a, b), lambda i: (0, 0))


def _f32(*shape):
    return jax.ShapeDtypeStruct(shape, jnp.float32)


def _apply_act(name, h, gate=None):
    if name == "none":
        return h
    if name == "tanh":
        return jnp.tanh(h)
    if name == "sigtanh":
        return jax.nn.sigmoid(jnp.tanh(h))
    if name == "tanhtanh":
        return jnp.tanh(jnp.tanh(h))
    if name == "gate_tanh":
        return gate * jnp.tanh(h)
    raise ValueError(name)


def _mm(a, b):
    return jax.lax.dot_general(a, b, (((1,), (0,)), ((), ())),
                               preferred_element_type=jnp.float32)


def _uz_specs(widths):
    return [_rows3(64) for _ in widths] + [_rows(64) for _ in widths]


def _combine(refs, K, fill, widths):
    parts = []
    for k, w in enumerate(widths):
        u = refs[k]
        z = refs[K + k]
        parts.append(u[0][:, :w] + u[1][:, :w] + fill * z[:, :w])
    return parts[0] if K == 1 else jnp.concatenate(parts, axis=1)


def _emit_chunks(zz, o_refs, widths):
    off = 0
    for k, w in enumerate(widths):
        if w == 64:
            o_refs[k][...] = zz[:, off:off + w]
        else:
            o_refs[k][:, :w] = zz[:, off:off + w]
        off += w


def _zout(F):
    widths = _chunkw(F)
    return (tuple(_rows(64) for _ in widths),
            tuple(_f32(N, 64) for _ in widths))


def _post_mm(us, zs, dinv, fill, W, b, act):
    F, G = W.shape
    widths = _chunkw(F)
    K = len(widths)

    def body(*refs):
        d_ref, w_ref, b_ref, o_ref = refs[2 * K], refs[2 * K + 1], \
            refs[2 * K + 2], refs[2 * K + 3]
        y = d_ref[...] * _combine(refs, K, fill, widths)
        h = _mm(y, w_ref[...]) + b_ref[...]
        o_ref[...] = _apply_act(act, h)

    return pl.pallas_call(
        body, grid=(GRID,),
        in_specs=_uz_specs(widths) + [_rows(1), _whole(F, G), _whole(1, G)],
        out_specs=_rows(G), out_shape=_f32(N, G),
    )(*us, *zs, dinv, W, b.reshape(1, G))


def _post_mm_max(us, zs, dinv, fill, W, b):
    F, G = W.shape
    widths = _chunkw(F)
    K = len(widths)

    def body(*refs):
        d_ref, w_ref, b_ref, o_ref = refs[2 * K], refs[2 * K + 1], \
            refs[2 * K + 2], refs[2 * K + 3]
        y = d_ref[...] * _combine(refs, K, fill, widths)
        h = jnp.tanh(_mm(y, w_ref[...]) + b_ref[...])

        @pl.when(pl.program_id(0) == 0)
        def _():
            o_ref[...] = jnp.full((1, G), -jnp.inf, jnp.float32)

        o_ref[...] = jnp.maximum(o_ref[...],
                                 jnp.max(h, axis=0, keepdims=True))

    return pl.pallas_call(
        body, grid=(GRID,),
        in_specs=_uz_specs(widths) + [_rows(1), _whole(F, G), _whole(1, G)],
        out_specs=pl.BlockSpec((1, G), lambda i: (0, 0)),
        out_shape=_f32(1, G),
    )(*us, *zs, dinv, W, b.reshape(1, G))


def _post_mm_split2z(us, zs, dinv, fill, W, b, dz):
    F, G = W.shape
    H = G // 2
    widths = _chunkw(F)
    K = len(widths)
    ow = _chunkw(H)

    def body(*refs):
        d_ref, w_ref, b_ref, dz_ref = refs[2 * K], refs[2 * K + 1], \
            refs[2 * K + 2], refs[2 * K + 3]
        o_refs = refs[2 * K + 4:]
        y = d_ref[...] * _combine(refs, K, fill, widths)
        h = jnp.tanh(_mm(y, w_ref[...]) + b_ref[...])
        zz = dz_ref[...] * h
        _emit_chunks(zz, o_refs, ow + ow)

    ospec, oshape = _zout(H)
    outs = pl.pallas_call(
        body, grid=(GRID,),
        in_specs=_uz_specs(widths) + [_rows(1), _whole(F, G), _whole(1, G),
                                      _rows(1)],
        out_specs=ospec + ospec, out_shape=oshape + oshape,
    )(*us, *zs, dinv, W, b.reshape(1, G), dz)
    nk = len(ow)
    return list(outs[:nk]), list(outs[nk:])


def _post_ew(us, zs, dinv, fill, b, act, gate=None, dz=None):
    F = b.shape[0]
    widths = _chunkw(F)
    K = len(widths)

    def body(*refs):
        d_ref, b_ref = refs[2 * K], refs[2 * K + 1]
        i = 2 * K + 2
        g_ref = None
        if gate is not None:
            g_ref = refs[i]
            i += 1
        dz_ref = refs[i] if dz is not None else None
        if dz is not None:
            i += 1
        o_ref = refs[i]
        y = d_ref[...] * _combine(refs, K, fill, widths) + b_ref[...]
        h = _apply_act(act, y, gate=None if g_ref is None else g_ref[...])
        o_ref[...] = h
        if dz is not None:
            _emit_chunks(dz_ref[...] * h, refs[i + 1:], _chunkw(F))

    in_specs = _uz_specs(widths) + [_rows(1), _whole(1, F)]
    ins = [*us, *zs, dinv, b.reshape(1, F)]
    if gate is not None:
        in_specs.append(_rows(F))
        ins.append(gate)
    if dz is not None:
        in_specs.append(_rows(1))
        ins.append(dz)
    if dz is None:
        out_specs, out_shape = _rows(F), _f32(N, F)
    else:
        zspec, zshape = _zout(F)
        out_specs = (_rows(F),) + zspec
        out_shape = (_f32(N, F),) + zshape
    outs = pl.pallas_call(body, grid=(GRID,), in_specs=in_specs,
                          out_specs=out_specs, out_shape=out_shape)(*ins)
    if dz is None:
        return outs
    return outs[0], list(outs[1:])


def _mm_scale(X, W, dinv):
    F, G = W.shape
    ow = _chunkw(G)

    def body(*refs):
        x_ref, w_ref, d_ref = refs[0], refs[1], refs[2]
        zz = d_ref[...] * _mm(x_ref[...], w_ref[...])
        _emit_chunks(zz, refs[3:], ow)

    ospec, oshape = _zout(G)
    outs = pl.pallas_call(
        body, grid=(GRID,),
        in_specs=[_rows(F), _whole(F, G), _rows(1)],
        out_specs=ospec, out_shape=oshape,
    )(X, W, dinv)
    return list(outs) if isinstance(outs, (tuple, list)) else [outs]


def _mm2_scale(Xa, Xb, Wa, Wb, dinv):
    Fa, G = Wa.shape
    Fb = Wb.shape[0]
    ow = _chunkw(G)

    def body(*refs):
        xa_ref, xb_ref, wa_ref, wb_ref, d_ref = refs[:5]
        zz = d_ref[...] * (_mm(xa_ref[...], wa_ref[...]) +
                           _mm(xb_ref[...], wb_ref[...]))
        _emit_chunks(zz, refs[5:], ow)

    ospec, oshape = _zout(G)
    outs = pl.pallas_call(
        body, grid=(GRID,),
        in_specs=[_rows(Fa), _rows(Fb), _whole(Fa, G), _whole(Fb, G),
                  _rows(1)],
        out_specs=ospec, out_shape=oshape,
    )(Xa, Xb, Wa, Wb, dinv)
    return list(outs)


def _scale3(a, b_, c, dinv):
    F = a.shape[1]

    def body(a_ref, b_ref, c_ref, d_ref, o_ref):
        zz = d_ref[...] * (a_ref[...] * b_ref[...] + c_ref[...])
        _emit_chunks(zz, [o_ref], [F])

    return [pl.pallas_call(
        body, grid=(GRID,),
        in_specs=[_rows(F), _rows(F), _rows(F), _rows(1)],
        out_specs=_rows(64), out_shape=_f32(N, 64),
    )(a, b_, c, dinv)]


def _pre7(i2, x, m, W7a, W7b, W7c, dinv):
    G = W7a.shape[1]
    Fm = W7a.shape[0]
    F2 = W7b.shape[0]
    Fx = W7c.shape[0]

    def body(i2_ref, x_ref, m_ref, wa_ref, wb_ref, wc_ref, d_ref, o_ref):
        row = _mm(m_ref[...], wa_ref[...])
        h = _mm(i2_ref[...], wb_ref[...]) + _mm(x_ref[...], wc_ref[...]) + row
        _emit_chunks(d_ref[...] * h, [o_ref], [G])

    return [pl.pallas_call(
        body, grid=(GRID,),
        in_specs=[_rows(F2), _rows(Fx), _whole(1, Fm), _whole(Fm, G),
                  _whole(F2, G), _whole(Fx, G), _rows(1)],
        out_specs=_rows(64), out_shape=_f32(N, 64),
    )(i2, x, m, W7a, W7b, W7c, dinv)]


def _dinv_calc(u_ones):
    def body(u_ref, o1_ref, o2_ref):
        deg = u_ref[0, :, 0:1] + u_ref[1, :, 0:1]
        o1_ref[...] = jax.lax.rsqrt(deg + 1.0)
        o2_ref[...] = jax.lax.rsqrt(deg + 2.0)

    return pl.pallas_call(
        body, grid=(GRID,),
        in_specs=[_rows3(64)],
        out_specs=(_rows(1), _rows(1)),
        out_shape=(_f32(N, 1), _f32(N, 1)),
    )(u_ones)


def _gcn_out(H, p, dinv, fill, act, edges, dz=None):
    zs = _mm_scale(H, p["W"], dinv)
    us = _segsum(zs, edges)
    return _post_ew(us, zs, dinv, fill, p["b"], act, dz=dz)


def _inception(p, xx, zs_x, us_x, dinv, edges, act, gate=None, dz=None):
    d = p["conv1"]["W"].shape[0]
    W13 = jnp.concatenate([p["conv1"]["W"], p["conv3"]["W"]], axis=1)
    b13 = jnp.concatenate([p["conv1"]["b"], p["conv3"]["b"]])
    zs1, zs2 = _post_mm_split2z(us_x, zs_x, dinv, 1.0, W13, b13, dinv)
    us1 = _segsum(zs1, edges)
    m = _post_mm_max(us1, zs1, dinv, 1.0, p["conv2"]["W"], p["conv2"]["b"])
    us2 = _segsum(zs2, edges)
    i2 = _post_mm(us2, zs2, dinv, 1.0, p["conv4"]["W"], p["conv4"]["b"],
                  "tanh")
    W7 = p["conv7"]["W"]
    zs7 = _pre7(i2, xx, m, W7[:4 * d], W7[4 * d:8 * d], W7[8 * d:], dinv)
    us7 = _segsum(zs7, edges)
    return _post_ew(us7, zs7, dinv, 1.0, p["conv7"]["b"], act, gate=gate,
                    dz=dz)


def _lstm(p, xin, dinv, edges):
    xx, zs_x = _gcn_out(xin, p["conv1"], dinv, 1.0, "none", edges, dz=dinv)
    us_x = _segsum(zs_x, edges)
    f1, zf1 = _inception(p["inc1"], xx, zs_x, us_x, dinv, edges, "tanh",
                         dz=dinv)
    uf1 = _segsum(zf1, edges)
    f = _inception(p["inc2"], f1, zf1, uf1, dinv, edges, "gate_tanh",
                   gate=xx)
    i3, zi3 = _inception(p["inc3"], xx, zs_x, us_x, dinv, edges, "tanh",
                         dz=dinv)
    ui3 = _segsum(zi3, edges)
    s1 = _inception(p["inc4"], i3, zi3, ui3, dinv, edges, "sigtanh")
    i5, zi5 = _inception(p["inc5"], xx, zs_x, us_x, dinv, edges, "tanh",
                         dz=dinv)
    ui5 = _segsum(zi5, edges)
    t2 = _inception(p["inc6"], i5, zi5, ui5, dinv, edges, "tanhtanh")
    z_i = _scale3(s1, t2, f, dinv)
    u_i = _segsum(z_i, edges)
    return _post_mm(u_i, z_i, dinv, 1.0, p["conv2"]["W"], p["conv2"]["b"],
                    "tanh")


def kernel(x, adj_t, batch, params):
    src = adj_t[0].astype(jnp.int32)
    dst = adj_t[1].astype(jnp.int32)
    pad = EP - E
    ar = jnp.arange(pad, dtype=jnp.int32)
    src_p = jnp.concatenate([src, (ar * 97) % N])
    dst_p = jnp.concatenate([dst, N + (ar % (NACC - N))])
    perm = jnp.argsort(src_p)
    src_p = src_p[perm]
    dst_p = dst_p[perm]
    src3 = src_p.reshape(NTILES, ET // 512, 4, 128)
    dst3 = dst_p.reshape(NTILES, ET // 512, 4, 128)
    zeros = jnp.zeros((NACC, 64), jnp.float32)
    edges = (src3, dst3, zeros)

    ones_z = jnp.ones((N, 64), jnp.float32)
    u_ones = _segsum([ones_z], edges)[0]
    dinv1, dinv2 = _dinv_calc(u_ones)

    h1 = _lstm(params["lstm1"], x, dinv1, edges)
    h2 = _lstm(params["lstm2"], h1, dinv1, edges)

    W = params["conv1"]["W"]
    zs = _mm2_scale(x, h2, W[:128], W[128:], dinv2)
    us = _segsum(zs, edges)
    h = _post_ew(us, zs, dinv2, 2.0, params["conv1"]["b"], "tanh")
    h = _gcn_out(h, params["conv2"], dinv2, 2.0, "tanh", edges)
    h = _gcn_out(h, params["conv3"], dinv2, 2.0, "tanh", edges)
    h, z4 = _gcn_out(h, params["conv4"], dinv2, 2.0, "tanh", edges, dz=dinv2)
    u5 = _segsum(z4, edges)
    return _post_mm(u5, z4, dinv2, 2.0, params["convOut"]["W"],
                    params["convOut"]["b"], "none")


# speedup vs baseline: 1.8995x; 1.8995x over previous
import functools

import jax
import jax.numpy as jnp
from jax import lax
from jax.experimental import pallas as pl
from jax.experimental.pallas import tpu as pltpu
from jax.experimental.pallas import tpu_sc as plsc

N = 10000
E = 320000
NTILES = 32
EP = 327680
ET = EP // NTILES
NACC = 10112
R = 2000
GRID = N // R


def _chunkw(F):
    return [32] if F == 32 else [64] * (F // 64)


@functools.cache
def _segsum_fn(F):
    NDMA = 4
    NPAIR = ET // 1024

    mesh = plsc.VectorSubcoreMesh(
        core_axis_name="c", subcore_axis_name="s", num_cores=2)

    @functools.partial(
        pl.kernel,
        out_type=jax.ShapeDtypeStruct((2, N, F), jnp.float32),
        mesh=mesh,
        compiler_params=pltpu.CompilerParams(use_tc_tiling_on_sc=False),
        scratch_types=[
            pltpu.VMEM((4, 128), jnp.int32),
            pltpu.VMEM((4, 128), jnp.int32),
            pltpu.VMEM((4, 128), jnp.int32),
            pltpu.VMEM((4, 128), jnp.int32),
            pltpu.VMEM((512, F), jnp.float32),
            pltpu.VMEM((512, F), jnp.float32),
            pltpu.VMEM_SHARED((NACC, F), jnp.float32),
            pltpu.SemaphoreType.DMA,
            pltpu.SemaphoreType.DMA,
        ],
    )
    def seg(z_hbm, src_hbm, dst_hbm, zeros_hbm, u_hbm,
            srcA, dstA, srcB, dstB, rowsA, rowsB, acc, gsem, ssem):
        c = lax.axis_index("c")
        s = lax.axis_index("s")
        w = c * 16 + s
        pltpu.sync_copy(zeros_hbm.at[pl.ds(s * 632, 632)],
                        acc.at[pl.ds(s * 632, 632)])
        plsc.subcore_barrier()

        def gather_chunk(ci, sv, dv, rows):
            pltpu.sync_copy(src_hbm.at[w, ci], sv)
            pltpu.sync_copy(dst_hbm.at[w, ci], dv)
            return [pltpu.async_copy(z_hbm.at[sv.at[j]],
                                     rows.at[pl.ds(j * 128, 128)], gsem)
                    for j in range(NDMA)]

        def scatter_chunk(dv, rows):
            for j in range(NDMA):
                pltpu.make_async_copy(rows.at[pl.ds(j * 128, 128)],
                                      acc.at[dv.at[j]], ssem).start(add=True)

        def drain_scatters(dv, rows):
            for j in range(NDMA):
                pltpu.make_async_copy(rows.at[pl.ds(j * 128, 128)],
                                      acc.at[dv.at[j]], ssem).wait()

        def body(i, carry):
            ga = gather_chunk(2 * i, srcA, dstA, rowsA)

            @pl.when(i > 0)
            def _():
                drain_scatters(dstB, rowsB)

            for cp in ga:
                cp.wait()
            scatter_chunk(dstA, rowsA)
            gb = gather_chunk(2 * i + 1, srcB, dstB, rowsB)
            for cp in gb:
                cp.wait()
            drain_scatters(dstA, rowsA)
            scatter_chunk(dstB, rowsB)
            return carry

        lax.fori_loop(0, NPAIR, body, 0)
        drain_scatters(dstB, rowsB)
        plsc.subcore_barrier()
        pltpu.sync_copy(acc.at[pl.ds(s * 624, 624)],
                        u_hbm.at[c, pl.ds(s * 624, 624)])

        @pl.when(s == 15)
        def _():
            pltpu.sync_copy(acc.at[pl.ds(9984, 16)],
                            u_hbm.at[c, pl.ds(9984, 16)])

    return seg


def _segsum(zs, edges):
    src3, dst3, zeros = edges
    return [_segsum_fn(z.shape[1])(z, src3, dst3, zeros[:, :z.shape[1]])
            for z in zs]


def _rows(F):
    return pl.BlockSpec((R, F), lambda i: (i, 0))


def _rows3(F):
    return pl.BlockSpec((2, R, F), lambda i: (0, i, 0))


def _whole(a, b):
    return pl.BlockSpec((a, b), lambda i: (0, 0))


def _f32(*shape):
    return jax.ShapeDtypeStruct(shape, jnp.float32)


def _apply_act(name, h, gate=None):
    if name == "none":
        return h
    if name == "tanh":
        return jnp.tanh(h)
    if name == "sigtanh":
        return jax.nn.sigmoid(jnp.tanh(h))
    if name == "tanhtanh":
        return jnp.tanh(jnp.tanh(h))
    if name == "gate_tanh":
        return gate * jnp.tanh(h)
    raise ValueError(name)


def _mm(a, b):
    return jax.lax.dot_general(a, b, (((1,), (0,)), ((), ())),
                               preferred_element_type=jnp.float32)


def _uz_specs(widths):
    return [_rows3(64) for _ in widths] + [_rows(64) for _ in widths]


def _combine(refs, K, fill, widths):
    parts = []
    for k, w in enumerate(widths):
        u = refs[k]
        z = refs[K + k]
        parts.append(u[0][:, :w] + u[1][:, :w] + fill * z[:, :w])
    return parts[0] if K == 1 else jnp.concatenate(parts, axis=1)


def _emit_chunks(zz, o_refs, widths):
    off = 0
    for k, w in enumerate(widths):
        if w == 64:
            o_refs[k][...] = zz[:, off:off + w]
        else:
            o_refs[k][:, :w] = zz[:, off:off + w]
        off += w


def _zout(F):
    widths = _chunkw(F)
    return (tuple(_rows(64) for _ in widths),
            tuple(_f32(N, 64) for _ in widths))


def _post_mm(us, zs, dinv, fill, W, b, act):
    F, G = W.shape
    widths = _chunkw(F)
    K = len(widths)

    def body(*refs):
        d_ref, w_ref, b_ref, o_ref = refs[2 * K], refs[2 * K + 1], \
            refs[2 * K + 2], refs[2 * K + 3]
        y = d_ref[...] * _combine(refs, K, fill, widths)
        h = _mm(y, w_ref[...]) + b_ref[...]
        o_ref[...] = _apply_act(act, h)

    return pl.pallas_call(
        body, grid=(GRID,),
        in_specs=_uz_specs(widths) + [_rows(1), _whole(F, G), _whole(1, G)],
        out_specs=_rows(G), out_shape=_f32(N, G),
    )(*us, *zs, dinv, W, b.reshape(1, G))


def _post_mm_max(us, zs, dinv, fill, W, b):
    F, G = W.shape
    widths = _chunkw(F)
    K = len(widths)

    def body(*refs):
        d_ref, w_ref, b_ref, o_ref = refs[2 * K], refs[2 * K + 1], \
            refs[2 * K + 2], refs[2 * K + 3]
        y = d_ref[...] * _combine(refs, K, fill, widths)
        h = jnp.tanh(_mm(y, w_ref[...]) + b_ref[...])

        @pl.when(pl.program_id(0) == 0)
        def _():
            o_ref[...] = jnp.full((1, G), -jnp.inf, jnp.float32)

        o_ref[...] = jnp.maximum(o_ref[...],
                                 jnp.max(h, axis=0, keepdims=True))

    return pl.pallas_call(
        body, grid=(GRID,),
        in_specs=_uz_specs(widths) + [_rows(1), _whole(F, G), _whole(1, G)],
        out_specs=pl.BlockSpec((1, G), lambda i: (0, 0)),
        out_shape=_f32(1, G),
    )(*us, *zs, dinv, W, b.reshape(1, G))


def _post_mm_split2z(us, zs, dinv, fill, W, b, dz):
    F, G = W.shape
    H = G // 2
    widths = _chunkw(F)
    K = len(widths)
    ow = _chunkw(H)

    def body(*refs):
        d_ref, w_ref, b_ref, dz_ref = refs[2 * K], refs[2 * K + 1], \
            refs[2 * K + 2], refs[2 * K + 3]
        o_refs = refs[2 * K + 4:]
        y = d_ref[...] * _combine(refs, K, fill, widths)
        h = jnp.tanh(_mm(y, w_ref[...]) + b_ref[...])
        zz = dz_ref[...] * h
        _emit_chunks(zz, o_refs, ow + ow)

    ospec, oshape = _zout(H)
    outs = pl.pallas_call(
        body, grid=(GRID,),
        in_specs=_uz_specs(widths) + [_rows(1), _whole(F, G), _whole(1, G),
                                      _rows(1)],
        out_specs=ospec + ospec, out_shape=oshape + oshape,
    )(*us, *zs, dinv, W, b.reshape(1, G), dz)
    nk = len(ow)
    return list(outs[:nk]), list(outs[nk:])


def _post_ew(us, zs, dinv, fill, b, act, gate=None, dz=None):
    F = b.shape[0]
    widths = _chunkw(F)
    K = len(widths)

    def body(*refs):
        d_ref, b_ref = refs[2 * K], refs[2 * K + 1]
        i = 2 * K + 2
        g_ref = None
        if gate is not None:
            g_ref = refs[i]
            i += 1
        dz_ref = refs[i] if dz is not None else None
        if dz is not None:
            i += 1
        o_ref = refs[i]
        y = d_ref[...] * _combine(refs, K, fill, widths) + b_ref[...]
        h = _apply_act(act, y, gate=None if g_ref is None else g_ref[...])
        o_ref[...] = h
        if dz is not None:
            _emit_chunks(dz_ref[...] * h, refs[i + 1:], _chunkw(F))

    in_specs = _uz_specs(widths) + [_rows(1), _whole(1, F)]
    ins = [*us, *zs, dinv, b.reshape(1, F)]
    if gate is not None:
        in_specs.append(_rows(F))
        ins.append(gate)
    if dz is not None:
        in_specs.append(_rows(1))
        ins.append(dz)
    if dz is None:
        out_specs, out_shape = _rows(F), _f32(N, F)
    else:
        zspec, zshape = _zout(F)
        out_specs = (_rows(F),) + zspec
        out_shape = (_f32(N, F),) + zshape
    outs = pl.pallas_call(body, grid=(GRID,), in_specs=in_specs,
                          out_specs=out_specs, out_shape=out_shape)(*ins)
    if dz is None:
        return outs
    return outs[0], list(outs[1:])


def _mm_scale(X, W, dinv):
    F, G = W.shape
    ow = _chunkw(G)

    def body(*refs):
        x_ref, w_ref, d_ref = refs[0], refs[1], refs[2]
        zz = d_ref[...] * _mm(x_ref[...], w_ref[...])
        _emit_chunks(zz, refs[3:], ow)

    ospec, oshape = _zout(G)
    outs = pl.pallas_call(
        body, grid=(GRID,),
        in_specs=[_rows(F), _whole(F, G), _rows(1)],
        out_specs=ospec, out_shape=oshape,
    )(X, W, dinv)
    return list(outs) if isinstance(outs, (tuple, list)) else [outs]


def _mm2_scale(Xa, Xb, Wa, Wb, dinv):
    Fa, G = Wa.shape
    Fb = Wb.shape[0]
    ow = _chunkw(G)

    def body(*refs):
        xa_ref, xb_ref, wa_ref, wb_ref, d_ref = refs[:5]
        zz = d_ref[...] * (_mm(xa_ref[...], wa_ref[...]) +
                           _mm(xb_ref[...], wb_ref[...]))
        _emit_chunks(zz, refs[5:], ow)

    ospec, oshape = _zout(G)
    outs = pl.pallas_call(
        body, grid=(GRID,),
        in_specs=[_rows(Fa), _rows(Fb), _whole(Fa, G), _whole(Fb, G),
                  _rows(1)],
        out_specs=ospec, out_shape=oshape,
    )(Xa, Xb, Wa, Wb, dinv)
    return list(outs)


def _scale3(a, b_, c, dinv):
    F = a.shape[1]

    def body(a_ref, b_ref, c_ref, d_ref, o_ref):
        zz = d_ref[...] * (a_ref[...] * b_ref[...] + c_ref[...])
        _emit_chunks(zz, [o_ref], [F])

    return [pl.pallas_call(
        body, grid=(GRID,),
        in_specs=[_rows(F), _rows(F), _rows(F), _rows(1)],
        out_specs=_rows(64), out_shape=_f32(N, 64),
    )(a, b_, c, dinv)]


def _pre7(i2, x, m, W7a, W7b, W7c, dinv):
    G = W7a.shape[1]
    Fm = W7a.shape[0]
    F2 = W7b.shape[0]
    Fx = W7c.shape[0]

    def body(i2_ref, x_ref, m_ref, wa_ref, wb_ref, wc_ref, d_ref, o_ref):
        row = _mm(m_ref[...], wa_ref[...])
        h = _mm(i2_ref[...], wb_ref[...]) + _mm(x_ref[...], wc_ref[...]) + row
        _emit_chunks(d_ref[...] * h, [o_ref], [G])

    return [pl.pallas_call(
        body, grid=(GRID,),
        in_specs=[_rows(F2), _rows(Fx), _whole(1, Fm), _whole(Fm, G),
                  _whole(F2, G), _whole(Fx, G), _rows(1)],
        out_specs=_rows(64), out_shape=_f32(N, 64),
    )(i2, x, m, W7a, W7b, W7c, dinv)]


def _dinv_calc(u_ones):
    def body(u_ref, o1_ref, o2_ref):
        deg = u_ref[0, :, 0:1] + u_ref[1, :, 0:1]
        o1_ref[...] = jax.lax.rsqrt(deg + 1.0)
        o2_ref[...] = jax.lax.rsqrt(deg + 2.0)

    return pl.pallas_call(
        body, grid=(GRID,),
        in_specs=[_rows3(64)],
        out_specs=(_rows(1), _rows(1)),
        out_shape=(_f32(N, 1), _f32(N, 1)),
    )(u_ones)


def _gcn_out(H, p, dinv, fill, act, edges, dz=None):
    zs = _mm_scale(H, p["W"], dinv)
    us = _segsum(zs, edges)
    return _post_ew(us, zs, dinv, fill, p["b"], act, dz=dz)


def _inception(p, xx, zs_x, us_x, dinv, edges, act, gate=None, dz=None):
    d = p["conv1"]["W"].shape[0]
    W13 = jnp.concatenate([p["conv1"]["W"], p["conv3"]["W"]], axis=1)
    b13 = jnp.concatenate([p["conv1"]["b"], p["conv3"]["b"]])
    zs1, zs2 = _post_mm_split2z(us_x, zs_x, dinv, 1.0, W13, b13, dinv)
    us1 = _segsum(zs1, edges)
    m = _post_mm_max(us1, zs1, dinv, 1.0, p["conv2"]["W"], p["conv2"]["b"])
    us2 = _segsum(zs2, edges)
    i2 = _post_mm(us2, zs2, dinv, 1.0, p["conv4"]["W"], p["conv4"]["b"],
                  "tanh")
    W7 = p["conv7"]["W"]
    zs7 = _pre7(i2, xx, m, W7[:4 * d], W7[4 * d:8 * d], W7[8 * d:], dinv)
    us7 = _segsum(zs7, edges)
    return _post_ew(us7, zs7, dinv, 1.0, p["conv7"]["b"], act, gate=gate,
                    dz=dz)


def _lstm(p, xin, dinv, edges):
    xx, zs_x = _gcn_out(xin, p["conv1"], dinv, 1.0, "none", edges, dz=dinv)
    us_x = _segsum(zs_x, edges)
    f1, zf1 = _inception(p["inc1"], xx, zs_x, us_x, dinv, edges, "tanh",
                         dz=dinv)
    uf1 = _segsum(zf1, edges)
    f = _inception(p["inc2"], f1, zf1, uf1, dinv, edges, "gate_tanh",
                   gate=xx)
    i3, zi3 = _inception(p["inc3"], xx, zs_x, us_x, dinv, edges, "tanh",
                         dz=dinv)
    ui3 = _segsum(zi3, edges)
    s1 = _inception(p["inc4"], i3, zi3, ui3, dinv, edges, "sigtanh")
    i5, zi5 = _inception(p["inc5"], xx, zs_x, us_x, dinv, edges, "tanh",
                         dz=dinv)
    ui5 = _segsum(zi5, edges)
    t2 = _inception(p["inc6"], i5, zi5, ui5, dinv, edges, "tanhtanh")
    z_i = _scale3(s1, t2, f, dinv)
    u_i = _segsum(z_i, edges)
    return _post_mm(u_i, z_i, dinv, 1.0, p["conv2"]["W"], p["conv2"]["b"],
                    "tanh")


def kernel(x, adj_t, batch, params):
    src = adj_t[0].astype(jnp.int32)
    dst = adj_t[1].astype(jnp.int32)
    pad = EP - E
    ar = jnp.arange(pad, dtype=jnp.int32)
    src_p = jnp.concatenate([src, (ar * 97) % N])
    dst_p = jnp.concatenate([dst, N + (ar % (NACC - N))])
    src3 = src_p.reshape(NTILES, ET // 512, 4, 128)
    dst3 = dst_p.reshape(NTILES, ET // 512, 4, 128)
    zeros = jnp.zeros((NACC, 64), jnp.float32)
    edges = (src3, dst3, zeros)

    ones_z = jnp.ones((N, 64), jnp.float32)
    u_ones = _segsum([ones_z], edges)[0]
    dinv1, dinv2 = _dinv_calc(u_ones)

    h1 = _lstm(params["lstm1"], x, dinv1, edges)
    h2 = _lstm(params["lstm2"], h1, dinv1, edges)

    W = params["conv1"]["W"]
    zs = _mm2_scale(x, h2, W[:128], W[128:], dinv2)
    us = _segsum(zs, edges)
    h = _post_ew(us, zs, dinv2, 2.0, params["conv1"]["b"], "tanh")
    h = _gcn_out(h, params["conv2"], dinv2, 2.0, "tanh", edges)
    h = _gcn_out(h, params["conv3"], dinv2, 2.0, "tanh", edges)
    h, z4 = _gcn_out(h, params["conv4"], dinv2, 2.0, "tanh", edges, dz=dinv2)
    u5 = _segsum(z4, edges)
    return _post_mm(u5, z4, dinv2, 2.0, params["convOut"]["W"],
                    params["convOut"]["b"], "none")


# speedup vs baseline: 1.9965x; 1.0510x over previous
import functools

import jax
import jax.numpy as jnp
from jax import lax
from jax.experimental import pallas as pl
from jax.experimental.pallas import tpu as pltpu
from jax.experimental.pallas import tpu_sc as plsc

N = 10000
E = 320000
NTILES = 32
EP = 327680
ET = EP // NTILES
NACC = 10112
R = 2000
GRID = N // R


def _chunkw(F):
    return [32] if F == 32 else [64] * (F // 64)


@functools.cache
def _segsum_fn(F):
    NDMA = 4
    NPAIR = ET // 1024

    mesh = plsc.VectorSubcoreMesh(
        core_axis_name="c", subcore_axis_name="s", num_cores=2)

    @functools.partial(
        pl.kernel,
        out_type=jax.ShapeDtypeStruct((2, N, F), jnp.float32),
        mesh=mesh,
        compiler_params=pltpu.CompilerParams(use_tc_tiling_on_sc=False),
        scratch_types=[
            pltpu.VMEM((4, 128), jnp.int32),
            pltpu.VMEM((4, 128), jnp.int32),
            pltpu.VMEM((4, 128), jnp.int32),
            pltpu.VMEM((4, 128), jnp.int32),
            pltpu.VMEM((512, F), jnp.float32),
            pltpu.VMEM((512, F), jnp.float32),
            pltpu.VMEM_SHARED((NACC, F), jnp.float32),
            pltpu.SemaphoreType.DMA,
            pltpu.SemaphoreType.DMA,
        ],
    )
    def seg(z_hbm, src_hbm, dst_hbm, zeros_hbm, u_hbm,
            srcA, dstA, srcB, dstB, rowsA, rowsB, acc, gsem, ssem):
        c = lax.axis_index("c")
        s = lax.axis_index("s")
        w = c * 16 + s
        pltpu.sync_copy(zeros_hbm.at[pl.ds(s * 632, 632)],
                        acc.at[pl.ds(s * 632, 632)])
        plsc.subcore_barrier()

        def gather_chunk(ci, sv, dv, rows):
            pltpu.sync_copy(src_hbm.at[w, ci], sv)
            pltpu.sync_copy(dst_hbm.at[w, ci], dv)
            return [pltpu.async_copy(z_hbm.at[sv.at[j]],
                                     rows.at[pl.ds(j * 128, 128)], gsem)
                    for j in range(NDMA)]

        def scatter_chunk(dv, rows):
            for j in range(NDMA):
                pltpu.make_async_copy(rows.at[pl.ds(j * 128, 128)],
                                      acc.at[dv.at[j]], ssem).start(add=True)

        def drain_scatters(dv, rows):
            for j in range(NDMA):
                pltpu.make_async_copy(rows.at[pl.ds(j * 128, 128)],
                                      acc.at[dv.at[j]], ssem).wait()

        def body(i, carry):
            ga = gather_chunk(2 * i, srcA, dstA, rowsA)

            @pl.when(i > 0)
            def _():
                drain_scatters(dstB, rowsB)

            for cp in ga:
                cp.wait()
            scatter_chunk(dstA, rowsA)
            gb = gather_chunk(2 * i + 1, srcB, dstB, rowsB)
            for cp in gb:
                cp.wait()
            drain_scatters(dstA, rowsA)
            scatter_chunk(dstB, rowsB)
            return carry

        lax.fori_loop(0, NPAIR, body, 0)
        drain_scatters(dstB, rowsB)
        plsc.subcore_barrier()
        pltpu.sync_copy(acc.at[pl.ds(s * 624, 624)],
                        u_hbm.at[c, pl.ds(s * 624, 624)])

        @pl.when(s == 15)
        def _():
            pltpu.sync_copy(acc.at[pl.ds(9984, 16)],
                            u_hbm.at[c, pl.ds(9984, 16)])

    return seg


def _segsum(zs, edges):
    src3, dst3, zeros = edges
    return [_segsum_fn(z.shape[1])(z, src3, dst3, zeros[:, :z.shape[1]])
            for z in zs]


def _rows(F):
    return pl.BlockSpec((R, F), lambda i: (i, 0))


def _rows3(F):
    return pl.BlockSpec((2, R, F), lambda i: (0, i, 0))


def _whole(a, b):
    return pl.BlockSpec((a, b), lambda i: (0, 0))


def _f32(*shape):
    return jax.ShapeDtypeStruct(shape, jnp.float32)


def _apply_act(name, h, gate=None):
    if name == "none":
        return h
    if name == "tanh":
        return jnp.tanh(h)
    if name == "sigtanh":
        return jax.nn.sigmoid(jnp.tanh(h))
    if name == "tanhtanh":
        return jnp.tanh(jnp.tanh(h))
    if name == "gate_tanh":
        return gate * jnp.tanh(h)
    raise ValueError(name)


def _mm(a, b):
    return jax.lax.dot_general(a, b, (((1,), (0,)), ((), ())),
                               preferred_element_type=jnp.float32)


def _uz_specs(widths):
    return [_rows3(w) for w in widths] + [_rows(w) for w in widths]


def _combine(refs, K, fill, widths):
    parts = []
    for k, w in enumerate(widths):
        u = refs[k]
        z = refs[K + k]
        parts.append(u[0][:, :w] + u[1][:, :w] + fill * z[:, :w])
    return parts[0] if K == 1 else jnp.concatenate(parts, axis=1)


def _emit_chunks(zz, o_refs, widths):
    off = 0
    for k, w in enumerate(widths):
        o_refs[k][...] = zz[:, off:off + w]
        off += w


def _zout(F):
    widths = _chunkw(F)
    return (tuple(_rows(w) for w in widths),
            tuple(_f32(N, w) for w in widths))


def _post_mm(us, zs, dinv, fill, W, b, act):
    F, G = W.shape
    widths = _chunkw(F)
    K = len(widths)

    def body(*refs):
        d_ref, w_ref, b_ref, o_ref = refs[2 * K], refs[2 * K + 1], \
            refs[2 * K + 2], refs[2 * K + 3]
        y = d_ref[...] * _combine(refs, K, fill, widths)
        h = _mm(y, w_ref[...]) + b_ref[...]
        o_ref[...] = _apply_act(act, h)

    return pl.pallas_call(
        body, grid=(GRID,),
        in_specs=_uz_specs(widths) + [_rows(1), _whole(F, G), _whole(1, G)],
        out_specs=_rows(G), out_shape=_f32(N, G),
    )(*us, *zs, dinv, W, b.reshape(1, G))


def _post_mm_max(us, zs, dinv, fill, W, b):
    F, G = W.shape
    widths = _chunkw(F)
    K = len(widths)

    def body(*refs):
        d_ref, w_ref, b_ref, o_ref = refs[2 * K], refs[2 * K + 1], \
            refs[2 * K + 2], refs[2 * K + 3]
        y = d_ref[...] * _combine(refs, K, fill, widths)
        h = jnp.tanh(_mm(y, w_ref[...]) + b_ref[...])

        @pl.when(pl.program_id(0) == 0)
        def _():
            o_ref[...] = jnp.full((1, G), -jnp.inf, jnp.float32)

        o_ref[...] = jnp.maximum(o_ref[...],
                                 jnp.max(h, axis=0, keepdims=True))

    return pl.pallas_call(
        body, grid=(GRID,),
        in_specs=_uz_specs(widths) + [_rows(1), _whole(F, G), _whole(1, G)],
        out_specs=pl.BlockSpec((1, G), lambda i: (0, 0)),
        out_shape=_f32(1, G),
    )(*us, *zs, dinv, W, b.reshape(1, G))


def _post_mm_split2z(us, zs, dinv, fill, W, b, dz):
    F, G = W.shape
    H = G // 2
    widths = _chunkw(F)
    K = len(widths)
    ow = _chunkw(H)

    def body(*refs):
        d_ref, w_ref, b_ref, dz_ref = refs[2 * K], refs[2 * K + 1], \
            refs[2 * K + 2], refs[2 * K + 3]
        o_refs = refs[2 * K + 4:]
        y = d_ref[...] * _combine(refs, K, fill, widths)
        h = jnp.tanh(_mm(y, w_ref[...]) + b_ref[...])
        zz = dz_ref[...] * h
        _emit_chunks(zz, o_refs, ow + ow)

    ospec, oshape = _zout(H)
    outs = pl.pallas_call(
        body, grid=(GRID,),
        in_specs=_uz_specs(widths) + [_rows(1), _whole(F, G), _whole(1, G),
                                      _rows(1)],
        out_specs=ospec + ospec, out_shape=oshape + oshape,
    )(*us, *zs, dinv, W, b.reshape(1, G), dz)
    nk = len(ow)
    return list(outs[:nk]), list(outs[nk:])


def _post_ew(us, zs, dinv, fill, b, act, gate=None, dz=None):
    F = b.shape[0]
    widths = _chunkw(F)
    K = len(widths)

    def body(*refs):
        d_ref, b_ref = refs[2 * K], refs[2 * K + 1]
        i = 2 * K + 2
        g_ref = None
        if gate is not None:
            g_ref = refs[i]
            i += 1
        dz_ref = refs[i] if dz is not None else None
        if dz is not None:
            i += 1
        o_ref = refs[i]
        y = d_ref[...] * _combine(refs, K, fill, widths) + b_ref[...]
        h = _apply_act(act, y, gate=None if g_ref is None else g_ref[...])
        o_ref[...] = h
        if dz is not None:
            _emit_chunks(dz_ref[...] * h, refs[i + 1:], _chunkw(F))

    in_specs = _uz_specs(widths) + [_rows(1), _whole(1, F)]
    ins = [*us, *zs, dinv, b.reshape(1, F)]
    if gate is not None:
        in_specs.append(_rows(F))
        ins.append(gate)
    if dz is not None:
        in_specs.append(_rows(1))
        ins.append(dz)
    if dz is None:
        out_specs, out_shape = _rows(F), _f32(N, F)
    else:
        zspec, zshape = _zout(F)
        out_specs = (_rows(F),) + zspec
        out_shape = (_f32(N, F),) + zshape
    outs = pl.pallas_call(body, grid=(GRID,), in_specs=in_specs,
                          out_specs=out_specs, out_shape=out_shape)(*ins)
    if dz is None:
        return outs
    return outs[0], list(outs[1:])


def _mm_scale(X, W, dinv):
    F, G = W.shape
    ow = _chunkw(G)

    def body(*refs):
        x_ref, w_ref, d_ref = refs[0], refs[1], refs[2]
        zz = d_ref[...] * _mm(x_ref[...], w_ref[...])
        _emit_chunks(zz, refs[3:], ow)

    ospec, oshape = _zout(G)
    outs = pl.pallas_call(
        body, grid=(GRID,),
        in_specs=[_rows(F), _whole(F, G), _rows(1)],
        out_specs=ospec, out_shape=oshape,
    )(X, W, dinv)
    return list(outs) if isinstance(outs, (tuple, list)) else [outs]


def _mm2_scale(Xa, Xb, Wa, Wb, dinv):
    Fa, G = Wa.shape
    Fb = Wb.shape[0]
    ow = _chunkw(G)

    def body(*refs):
        xa_ref, xb_ref, wa_ref, wb_ref, d_ref = refs[:5]
        zz = d_ref[...] * (_mm(xa_ref[...], wa_ref[...]) +
                           _mm(xb_ref[...], wb_ref[...]))
        _emit_chunks(zz, refs[5:], ow)

    ospec, oshape = _zout(G)
    outs = pl.pallas_call(
        body, grid=(GRID,),
        in_specs=[_rows(Fa), _rows(Fb), _whole(Fa, G), _whole(Fb, G),
                  _rows(1)],
        out_specs=ospec, out_shape=oshape,
    )(Xa, Xb, Wa, Wb, dinv)
    return list(outs)


def _scale3(a, b_, c, dinv):
    F = a.shape[1]

    def body(a_ref, b_ref, c_ref, d_ref, o_ref):
        zz = d_ref[...] * (a_ref[...] * b_ref[...] + c_ref[...])
        _emit_chunks(zz, [o_ref], [F])

    return [pl.pallas_call(
        body, grid=(GRID,),
        in_specs=[_rows(F), _rows(F), _rows(F), _rows(1)],
        out_specs=_rows(F), out_shape=_f32(N, F),
    )(a, b_, c, dinv)]


def _pre7(i2, x, m, W7a, W7b, W7c, dinv):
    G = W7a.shape[1]
    Fm = W7a.shape[0]
    F2 = W7b.shape[0]
    Fx = W7c.shape[0]

    def body(i2_ref, x_ref, m_ref, wa_ref, wb_ref, wc_ref, d_ref, o_ref):
        row = _mm(m_ref[...], wa_ref[...])
        h = _mm(i2_ref[...], wb_ref[...]) + _mm(x_ref[...], wc_ref[...]) + row
        _emit_chunks(d_ref[...] * h, [o_ref], [G])

    return [pl.pallas_call(
        body, grid=(GRID,),
        in_specs=[_rows(F2), _rows(Fx), _whole(1, Fm), _whole(Fm, G),
                  _whole(F2, G), _whole(Fx, G), _rows(1)],
        out_specs=_rows(G), out_shape=_f32(N, G),
    )(i2, x, m, W7a, W7b, W7c, dinv)]


def _dinv_calc(u_ones):
    def body(u_ref, o1_ref, o2_ref):
        deg = u_ref[0, :, 0:1] + u_ref[1, :, 0:1]
        o1_ref[...] = jax.lax.rsqrt(deg + 1.0)
        o2_ref[...] = jax.lax.rsqrt(deg + 2.0)

    return pl.pallas_call(
        body, grid=(GRID,),
        in_specs=[_rows3(32)],
        out_specs=(_rows(1), _rows(1)),
        out_shape=(_f32(N, 1), _f32(N, 1)),
    )(u_ones)


def _gcn_out(H, p, dinv, fill, act, edges, dz=None):
    zs = _mm_scale(H, p["W"], dinv)
    us = _segsum(zs, edges)
    return _post_ew(us, zs, dinv, fill, p["b"], act, dz=dz)


def _inception(p, xx, zs_x, us_x, dinv, edges, act, gate=None, dz=None):
    d = p["conv1"]["W"].shape[0]
    W13 = jnp.concatenate([p["conv1"]["W"], p["conv3"]["W"]], axis=1)
    b13 = jnp.concatenate([p["conv1"]["b"], p["conv3"]["b"]])
    zs1, zs2 = _post_mm_split2z(us_x, zs_x, dinv, 1.0, W13, b13, dinv)
    us1 = _segsum(zs1, edges)
    m = _post_mm_max(us1, zs1, dinv, 1.0, p["conv2"]["W"], p["conv2"]["b"])
    us2 = _segsum(zs2, edges)
    i2 = _post_mm(us2, zs2, dinv, 1.0, p["conv4"]["W"], p["conv4"]["b"],
                  "tanh")
    W7 = p["conv7"]["W"]
    zs7 = _pre7(i2, xx, m, W7[:4 * d], W7[4 * d:8 * d], W7[8 * d:], dinv)
    us7 = _segsum(zs7, edges)
    return _post_ew(us7, zs7, dinv, 1.0, p["conv7"]["b"], act, gate=gate,
                    dz=dz)


def _lstm(p, xin, dinv, edges):
    xx, zs_x = _gcn_out(xin, p["conv1"], dinv, 1.0, "none", edges, dz=dinv)
    us_x = _segsum(zs_x, edges)
    f1, zf1 = _inception(p["inc1"], xx, zs_x, us_x, dinv, edges, "tanh",
                         dz=dinv)
    uf1 = _segsum(zf1, edges)
    f = _inception(p["inc2"], f1, zf1, uf1, dinv, edges, "gate_tanh",
                   gate=xx)
    i3, zi3 = _inception(p["inc3"], xx, zs_x, us_x, dinv, edges, "tanh",
                         dz=dinv)
    ui3 = _segsum(zi3, edges)
    s1 = _inception(p["inc4"], i3, zi3, ui3, dinv, edges, "sigtanh")
    i5, zi5 = _inception(p["inc5"], xx, zs_x, us_x, dinv, edges, "tanh",
                         dz=dinv)
    ui5 = _segsum(zi5, edges)
    t2 = _inception(p["inc6"], i5, zi5, ui5, dinv, edges, "tanhtanh")
    z_i = _scale3(s1, t2, f, dinv)
    u_i = _segsum(z_i, edges)
    return _post_mm(u_i, z_i, dinv, 1.0, p["conv2"]["W"], p["conv2"]["b"],
                    "tanh")


def kernel(x, adj_t, batch, params):
    src = adj_t[0].astype(jnp.int32)
    dst = adj_t[1].astype(jnp.int32)
    pad = EP - E
    ar = jnp.arange(pad, dtype=jnp.int32)
    src_p = jnp.concatenate([src, (ar * 97) % N])
    dst_p = jnp.concatenate([dst, N + (ar % (NACC - N))])
    src3 = src_p.reshape(NTILES, ET // 512, 4, 128)
    dst3 = dst_p.reshape(NTILES, ET // 512, 4, 128)
    zeros = jnp.zeros((NACC, 64), jnp.float32)
    edges = (src3, dst3, zeros)

    ones_z = jnp.ones((N, 32), jnp.float32)
    u_ones = _segsum([ones_z], edges)[0]
    dinv1, dinv2 = _dinv_calc(u_ones)

    h1 = _lstm(params["lstm1"], x, dinv1, edges)
    h2 = _lstm(params["lstm2"], h1, dinv1, edges)

    W = params["conv1"]["W"]
    zs = _mm2_scale(x, h2, W[:128], W[128:], dinv2)
    us = _segsum(zs, edges)
    h = _post_ew(us, zs, dinv2, 2.0, params["conv1"]["b"], "tanh")
    h = _gcn_out(h, params["conv2"], dinv2, 2.0, "tanh", edges)
    h = _gcn_out(h, params["conv3"], dinv2, 2.0, "tanh", edges)
    h, z4 = _gcn_out(h, params["conv4"], dinv2, 2.0, "tanh", edges, dz=dinv2)
    u5 = _segsum(z4, edges)
    return _post_mm(u5, z4, dinv2, 2.0, params["convOut"]["W"],
                    params["convOut"]["b"], "none")
